# baseline (speedup 1.0000x reference)
import functools

import jax
import jax.numpy as jnp
from jax import lax
from jax.experimental import pallas as pl
from jax.experimental.pallas import tpu as pltpu
from jax.experimental.pallas import tpu_sc as plsc

_LAMBDA_CENTER = 1.0
_LAMBDA_KL = 0.05
_LAMBDA_KL_WARMUP = 0.005
_KNN_K = 5
_SIGMA_S_INIT = 2.0
_SIGMA_S_FINAL = 0.8
_WARMUP_ITERS = 1000
_ANNEAL_ITERS = 3000
_PRIOR_DELTA_MIN = 0.5
_PRIOR_DELTA_MAX = 20.0
_LOG_SIGMA_MIN = -6.0
_LOG_SIGMA_MAX = 4.0
_BIG = 3.0e38

_NC, _NS, _L = 2, 16, 16
_NW = _NC * _NS
_CHUNK = _NW * _L
_MASK_PEN = 1.0e10


def _knn_sc(qx, qy, kx, ky, k):
    n_pad = qx.shape[0]
    m = kx.shape[0]
    rpt = n_pad // _NW
    nv = rpt // _L

    mesh = plsc.VectorSubcoreMesh(core_axis_name="c", subcore_axis_name="s",
                                  num_cores=_NC, num_subcores=_NS)

    @functools.partial(
        pl.kernel,
        out_type=jax.ShapeDtypeStruct((k * n_pad,), jnp.float32),
        mesh=mesh,
        scratch_types=[
            pltpu.VMEM((rpt,), jnp.float32),
            pltpu.VMEM((rpt,), jnp.float32),
            pltpu.VMEM((m,), jnp.float32),
            pltpu.VMEM((m,), jnp.float32),
            pltpu.VMEM((k * rpt,), jnp.float32),
        ],
    )
    def knn(qx_h, qy_h, kx_h, ky_h, out_h, qx_v, qy_v, kx_v, ky_v, out_v):
        wid = lax.axis_index("s") * _NC + lax.axis_index("c")
        base = wid * rpt
        pltpu.sync_copy(qx_h.at[pl.ds(base, rpt)], qx_v)
        pltpu.sync_copy(qy_h.at[pl.ds(base, rpt)], qy_v)
        pltpu.sync_copy(kx_h, kx_v)
        pltpu.sync_copy(ky_h, ky_v)
        for r in range(nv):
            qxv = qx_v[pl.ds(r * _L, _L)]
            qyv = qy_v[pl.ds(r * _L, _L)]
            init = tuple(jnp.full((_L,), _BIG, jnp.float32) for _ in range(k))

            def body(c, ms, qxv=qxv, qyv=qyv):
                kxv16 = kx_v[pl.ds(c * _L, _L)]
                kyv16 = ky_v[pl.ds(c * _L, _L)]
                for l in range(_L):
                    dx = qxv - kxv16[l]
                    dy = qyv - kyv16[l]
                    d2 = dx * dx + dy * dy
                    t = jnp.where(d2 < 1e-4, d2 + _MASK_PEN, d2)
                    out = []
                    for mm in ms:
                        out.append(jnp.minimum(mm, t))
                        t = jnp.maximum(mm, t)
                    ms = tuple(out)
                return ms

            ms = lax.fori_loop(0, m // _L, body, init)
            for jj in range(k):
                out_v[pl.ds(jj * rpt + r * _L, _L)] = ms[jj]
        for jj in range(k):
            pltpu.sync_copy(out_v.at[pl.ds(jj * rpt, rpt)],
                            out_h.at[pl.ds(jj * n_pad + base, rpt)])

    return knn(qx, qy, kx, ky)


def _loss_tail(q, mu, bls, pos, stride, d_i, sig_s, rmask):
    gt_delta = (q - pos) / stride
    diff = mu[:, 0:2] - gt_delta
    ad = jnp.abs(diff)
    sl1 = jnp.where(ad < 1.0, 0.5 * diff * diff, ad - 0.5)

    d_norm = jnp.clip(d_i / stride, _PRIOR_DELTA_MIN, _PRIOR_DELTA_MAX)
    sigma_c = jnp.maximum(d_norm, 1.0)
    mu_s = jnp.log(d_norm)

    log_sq = jnp.clip(bls, _LOG_SIGMA_MIN, _LOG_SIGMA_MAX)
    sigma_q = jnp.exp(log_sq)

    prior_mu = jnp.concatenate(
        [jnp.zeros_like(d_norm), jnp.zeros_like(d_norm), mu_s, mu_s], axis=1)
    prior_sigma = jnp.concatenate(
        [sigma_c, sigma_c,
         jnp.full_like(d_norm, 1.0) * sig_s,
         jnp.full_like(d_norm, 1.0) * sig_s], axis=1)
    sigma_p = jnp.clip(prior_sigma, 0.0001, None)

    dm = mu - prior_mu
    kl = (jnp.log(sigma_p / sigma_q)
          + (sigma_q * sigma_q + dm * dm) / (2.0 * sigma_p * sigma_p) - 0.5)

    s_center = jnp.sum(jnp.where(rmask, sl1[:, 0:1] + sl1[:, 1:2], 0.0))
    s_ckl = jnp.sum(jnp.where(rmask, kl[:, 0:1] + kl[:, 1:2], 0.0))
    s_skl = jnp.sum(jnp.where(rmask, kl[:, 2:3] + kl[:, 3:4], 0.0))
    return s_center, s_ckl, s_skl


def _accum_out(i, out_ref, s_center, s_ckl, s_skl):
    lane = jax.lax.broadcasted_iota(jnp.int32, (1, 128), 1)
    vec = (jnp.where(lane == 0, s_center, 0.0)
           + jnp.where(lane == 1, s_ckl, 0.0)
           + jnp.where(lane == 2, s_skl, 0.0))

    @pl.when(i == 0)
    def _():
        out_ref[...] = jnp.zeros_like(out_ref)

    out_ref[...] += vec


def _tc_body(mu_ref, bls_ref, pos_ref, stride_ref, gtc_ref, kx_ref, ky_ref,
             sig_ref, out_ref, *, m, k, n_valid):
    i = pl.program_id(0)

    q = gtc_ref[...]
    qx = q[:, 0:1]
    qy = q[:, 1:2]
    kx = kx_ref[...]
    ky = ky_ref[...]

    qn = qx * qx + qy * qy
    kn = kx * kx + ky * ky
    cross = qx * kx + qy * ky
    d2 = qn + kn - 2.0 * cross
    d = jnp.sqrt(jnp.clip(d2, 1e-12, None))
    d = jnp.where(d < 0.01, d + 1.0e8, d)

    colid = jax.lax.broadcasted_iota(jnp.int32, d.shape, 1)
    total = jnp.zeros((d.shape[0], 1), jnp.float32)
    for _ in range(k):
        mn = jnp.min(d, axis=1, keepdims=True)
        total = total + mn
        ism = d == mn
        first = jnp.min(jnp.where(ism, colid, m), axis=1, keepdims=True)
        d = jnp.where(colid == first, _BIG, d)
    d_i = total * (1.0 / k)

    br = q.shape[0]
    rowid = i * br + jax.lax.broadcasted_iota(jnp.int32, (br, 1), 0)
    rmask = rowid < n_valid
    s_center, s_ckl, s_skl = _loss_tail(
        q, mu_ref[...], bls_ref[...], pos_ref[...], stride_ref[...],
        d_i, sig_ref[0, 0], rmask)
    _accum_out(i, out_ref, s_center, s_ckl, s_skl)


def _sc_loss_body(mu_ref, bls_ref, pos_ref, stride_ref, gtc_ref, knn_ref,
                  sig_ref, out_ref, *, k, n_valid):
    i = pl.program_id(0)

    s5 = knn_ref[...]
    masked = s5 >= 1.0e9
    d2 = s5 - jnp.where(masked, _MASK_PEN, 0.0)
    d5 = (jnp.sqrt(jnp.clip(d2, 1e-12, None))
          + jnp.where(masked, 1.0e8, 0.0))
    d_i = jnp.sum(d5, axis=1, keepdims=True) * (1.0 / k)

    q = gtc_ref[...]
    br = q.shape[0]
    rowid = i * br + jax.lax.broadcasted_iota(jnp.int32, (br, 1), 0)
    rmask = rowid < n_valid
    s_center, s_ckl, s_skl = _loss_tail(
        q, mu_ref[...], bls_ref[...], pos_ref[...], stride_ref[...],
        d_i, sig_ref[0, 0], rmask)
    _accum_out(i, out_ref, s_center, s_ckl, s_skl)


def _pick_br(n):
    for br in (400, 256, 128, 64, 32, 16, 8):
        if n % br == 0:
            return br
    return n


def _pad_rows(a, n_to):
    n = a.shape[0]
    return jnp.pad(a, ((0, n_to - n),) + ((0, 0),) * (a.ndim - 1))


@jax.jit
def kernel(bbox_mu, bbox_log_sigma, pos_points, pos_strides, gt_centers,
           gt_centers_list, cur_iter):
    n = bbox_mu.shape[0]
    keys = gt_centers_list.reshape(-1, 2)
    m = keys.shape[0]
    k = min(_KNN_K, m - 1)

    ratio = jnp.clip((cur_iter - _WARMUP_ITERS) / max(_ANNEAL_ITERS, 1), 0.0, 1.0)
    eff_lkl = _LAMBDA_KL_WARMUP + ratio * (_LAMBDA_KL - _LAMBDA_KL_WARMUP)
    sigma_s = _SIGMA_S_INIT - ratio * (_SIGMA_S_INIT - _SIGMA_S_FINAL)
    sig_eff = jnp.maximum(sigma_s, 1.0).astype(jnp.float32).reshape(1, 1)

    stride_all = pos_strides.astype(jnp.float32).reshape(n, 1)

    _BR = 400
    nb_all = (n + _BR - 1) // _BR
    nb_tc = max(1, min(nb_all, int(round(n * 0.56 / _BR))))
    n_tc = min(nb_tc * _BR, n)
    n_sc = n - n_tc
    nb_b = (n_sc + _BR - 1) // _BR
    n_row_pad = nb_all * _BR

    def padr(a, cval=0.0):
        if n_row_pad == n:
            return a
        return jnp.pad(a, ((0, n_row_pad - n),) + ((0, 0),) * (a.ndim - 1),
                       constant_values=cval)

    mu_p = padr(bbox_mu)
    bls_p = padr(bbox_log_sigma)
    pos_p = padr(pos_points)
    str_p = padr(stride_all, 1.0)
    gtc_p = padr(gt_centers)

    kxT = keys[:, 0].reshape(1, m)
    kyT = keys[:, 1].reshape(1, m)

    if n_sc > 0:
        n_sc_pad = ((n_sc + _CHUNK - 1) // _CHUNK) * _CHUNK
        qx = jnp.pad(gt_centers[n_tc:, 0], (0, n_sc_pad - n_sc),
                     constant_values=1.0e9)
        qy = jnp.pad(gt_centers[n_tc:, 1], (0, n_sc_pad - n_sc),
                     constant_values=1.0e9)
        m_pad = ((m + _L - 1) // _L) * _L
        kx = jnp.pad(keys[:, 0], (0, m_pad - m), constant_values=1.0e8)
        ky = jnp.pad(keys[:, 1], (0, m_pad - m), constant_values=1.0e8)
        knn_s = _knn_sc(qx, qy, kx, ky, k)
        knn_rows = knn_s.reshape(k, n_sc_pad).T
        if nb_b * _BR > n_sc_pad:
            knn_rows = _pad_rows(knn_rows, nb_b * _BR)

    row_spec = lambda c: pl.BlockSpec((_BR, c), lambda i: (i, 0))
    full_spec = lambda r, c: pl.BlockSpec((r, c), lambda i: (0, 0))

    out_a = pl.pallas_call(
        functools.partial(_tc_body, m=m, k=k, n_valid=n_tc),
        grid=(nb_tc,),
        in_specs=[
            row_spec(4),
            row_spec(4),
            row_spec(2),
            row_spec(1),
            row_spec(2),
            full_spec(1, m),
            full_spec(1, m),
            full_spec(1, 1),
        ],
        out_specs=pl.BlockSpec((1, 128), lambda i: (0, 0)),
        out_shape=jax.ShapeDtypeStruct((1, 128), jnp.float32),
    )(mu_p, bls_p, pos_p, str_p, gtc_p, kxT, kyT, sig_eff)

    s_center = out_a[0, 0]
    s_ckl = out_a[0, 1]
    s_skl = out_a[0, 2]

    if n_sc > 0:
        tail_spec = lambda c: pl.BlockSpec((_BR, c), lambda i: (i + nb_tc, 0))
        out_b = pl.pallas_call(
            functools.partial(_sc_loss_body, k=k, n_valid=n_sc),
            grid=(nb_b,),
            in_specs=[
                tail_spec(4),
                tail_spec(4),
                tail_spec(2),
                tail_spec(1),
                tail_spec(2),
                pl.BlockSpec((_BR, k), lambda i: (i, 0)),
                pl.BlockSpec((1, 1), lambda i: (0, 0)),
            ],
            out_specs=pl.BlockSpec((1, 128), lambda i: (0, 0)),
            out_shape=jax.ShapeDtypeStruct((1, 128), jnp.float32),
        )(mu_p, bls_p, pos_p, str_p, gtc_p, knn_rows, sig_eff)
        s_center = s_center + out_b[0, 0]
        s_ckl = s_ckl + out_b[0, 1]
        s_skl = s_skl + out_b[0, 2]

    l_center = s_center / n
    center_kl = s_ckl / n
    scale_kl = s_skl / n
    l_kl = center_kl + ratio * scale_kl
    weighted_center = (_LAMBDA_CENTER * l_center).astype(jnp.float32)
    weighted_kl = (eff_lkl * l_kl).astype(jnp.float32)
    return (weighted_center, weighted_kl)


# speedup vs baseline: 1.0549x; 1.0549x over previous
import functools

import jax
import jax.numpy as jnp
from jax import lax
from jax.experimental import pallas as pl
from jax.experimental.pallas import tpu as pltpu
from jax.experimental.pallas import tpu_sc as plsc

_LAMBDA_CENTER = 1.0
_LAMBDA_KL = 0.05
_LAMBDA_KL_WARMUP = 0.005
_KNN_K = 5
_SIGMA_S_INIT = 2.0
_SIGMA_S_FINAL = 0.8
_WARMUP_ITERS = 1000
_ANNEAL_ITERS = 3000
_PRIOR_DELTA_MIN = 0.5
_PRIOR_DELTA_MAX = 20.0
_LOG_SIGMA_MIN = -6.0
_LOG_SIGMA_MAX = 4.0
_BIG = 3.0e38

_NC, _NS, _L = 2, 16, 16
_NW = _NC * _NS
_CHUNK = _NW * _L
_MASK_PEN = 1.0e10


def _knn_sc(qx, qy, kx, ky, k):
    n_pad = qx.shape[0]
    m = kx.shape[0]
    rpt = n_pad // _NW
    nv = rpt // _L

    mesh = plsc.VectorSubcoreMesh(core_axis_name="c", subcore_axis_name="s",
                                  num_cores=_NC, num_subcores=_NS)

    @functools.partial(
        pl.kernel,
        out_type=jax.ShapeDtypeStruct((k * n_pad,), jnp.float32),
        mesh=mesh,
        scratch_types=[
            pltpu.VMEM((rpt,), jnp.float32),
            pltpu.VMEM((rpt,), jnp.float32),
            pltpu.VMEM((m,), jnp.float32),
            pltpu.VMEM((m,), jnp.float32),
            pltpu.VMEM((k * rpt,), jnp.float32),
        ],
    )
    def knn(qx_h, qy_h, kx_h, ky_h, out_h, qx_v, qy_v, kx_v, ky_v, out_v):
        wid = lax.axis_index("s") * _NC + lax.axis_index("c")
        base = wid * rpt
        pltpu.sync_copy(qx_h.at[pl.ds(base, rpt)], qx_v)
        pltpu.sync_copy(qy_h.at[pl.ds(base, rpt)], qy_v)
        pltpu.sync_copy(kx_h, kx_v)
        pltpu.sync_copy(ky_h, ky_v)
        for r in range(nv):
            qxv = qx_v[pl.ds(r * _L, _L)]
            qyv = qy_v[pl.ds(r * _L, _L)]
            init = tuple(jnp.full((_L,), _BIG, jnp.float32) for _ in range(k))

            def body(c, ms, qxv=qxv, qyv=qyv):
                kxv16 = kx_v[pl.ds(c * _L, _L)]
                kyv16 = ky_v[pl.ds(c * _L, _L)]
                for l in range(_L):
                    dx = qxv - kxv16[l]
                    dy = qyv - kyv16[l]
                    d2 = dx * dx + dy * dy
                    t = jnp.where(d2 < 1e-4, d2 + _MASK_PEN, d2)
                    out = []
                    for mm in ms:
                        out.append(jnp.minimum(mm, t))
                        t = jnp.maximum(mm, t)
                    ms = tuple(out)
                return ms

            ms = lax.fori_loop(0, m // _L, body, init)
            for jj in range(k):
                out_v[pl.ds(jj * rpt + r * _L, _L)] = ms[jj]
        for jj in range(k):
            pltpu.sync_copy(out_v.at[pl.ds(jj * rpt, rpt)],
                            out_h.at[pl.ds(jj * n_pad + base, rpt)])

    return knn(qx, qy, kx, ky)


def _loss_tail(q, mu, bls, pos, stride, d_i, sig_s, rmask):
    gt_delta = (q - pos) / stride
    diff = mu[:, 0:2] - gt_delta
    ad = jnp.abs(diff)
    sl1 = jnp.where(ad < 1.0, 0.5 * diff * diff, ad - 0.5)

    d_norm = jnp.clip(d_i / stride, _PRIOR_DELTA_MIN, _PRIOR_DELTA_MAX)
    sigma_c = jnp.maximum(d_norm, 1.0)
    mu_s = jnp.log(d_norm)

    log_sq = jnp.clip(bls, _LOG_SIGMA_MIN, _LOG_SIGMA_MAX)
    sigma_q = jnp.exp(log_sq)

    prior_mu = jnp.concatenate(
        [jnp.zeros_like(d_norm), jnp.zeros_like(d_norm), mu_s, mu_s], axis=1)
    prior_sigma = jnp.concatenate(
        [sigma_c, sigma_c,
         jnp.full_like(d_norm, 1.0) * sig_s,
         jnp.full_like(d_norm, 1.0) * sig_s], axis=1)
    sigma_p = jnp.clip(prior_sigma, 0.0001, None)

    dm = mu - prior_mu
    kl = (jnp.log(sigma_p / sigma_q)
          + (sigma_q * sigma_q + dm * dm) / (2.0 * sigma_p * sigma_p) - 0.5)

    s_center = jnp.sum(jnp.where(rmask, sl1[:, 0:1] + sl1[:, 1:2], 0.0))
    s_ckl = jnp.sum(jnp.where(rmask, kl[:, 0:1] + kl[:, 1:2], 0.0))
    s_skl = jnp.sum(jnp.where(rmask, kl[:, 2:3] + kl[:, 3:4], 0.0))
    return s_center, s_ckl, s_skl


def _accum_out(i, out_ref, s_center, s_ckl, s_skl):
    lane = jax.lax.broadcasted_iota(jnp.int32, (1, 128), 1)
    vec = (jnp.where(lane == 0, s_center, 0.0)
           + jnp.where(lane == 1, s_ckl, 0.0)
           + jnp.where(lane == 2, s_skl, 0.0))

    @pl.when(i == 0)
    def _():
        out_ref[...] = jnp.zeros_like(out_ref)

    out_ref[...] += vec


def _tc_body(mu_ref, bls_ref, pos_ref, stride_ref, gtc_ref, keys_ref,
             sig_ref, out_ref, *, m, k, n_valid):
    i = pl.program_id(0)

    q = gtc_ref[...]
    qx = q[:, 0:1]
    qy = q[:, 1:2]
    kx = keys_ref[0:1, :]
    ky = keys_ref[1:2, :]

    qn = qx * qx + qy * qy
    kn = kx * kx + ky * ky
    cross = qx * kx + qy * ky
    d2 = qn + kn - 2.0 * cross
    t = jnp.where(d2 < 1e-4, d2 + _MASK_PEN, d2)

    total = jnp.zeros((t.shape[0], 1), jnp.float32)
    remaining = jnp.full((t.shape[0], 1), float(k), jnp.float32)
    for r in range(k):
        mn = jnp.min(t, axis=1, keepdims=True)
        msk = t == mn
        c = jnp.sum(msk, axis=1, keepdims=True).astype(jnp.float32)
        take = jnp.minimum(c, remaining)
        pen = mn >= 1.0e9
        mn2 = mn - jnp.where(pen, _MASK_PEN, 0.0)
        dval = (jnp.sqrt(jnp.clip(mn2, 1e-12, None))
                + jnp.where(pen, 1.0e8, 0.0))
        total = total + dval * take
        remaining = remaining - take
        if r < k - 1:
            t = jnp.where(msk, _BIG, t)
    d_i = total * (1.0 / k)

    br = q.shape[0]
    rowid = i * br + jax.lax.broadcasted_iota(jnp.int32, (br, 1), 0)
    rmask = rowid < n_valid
    s_center, s_ckl, s_skl = _loss_tail(
        q, mu_ref[...], bls_ref[...], pos_ref[...], stride_ref[...],
        d_i, sig_ref[0, 0], rmask)
    _accum_out(i, out_ref, s_center, s_ckl, s_skl)


def _sc_loss_body(mu_ref, bls_ref, pos_ref, stride_ref, gtc_ref, knn_ref,
                  sig_ref, out_ref, *, k, n_valid):
    i = pl.program_id(0)

    s5 = knn_ref[...]
    masked = s5 >= 1.0e9
    d2 = s5 - jnp.where(masked, _MASK_PEN, 0.0)
    d5 = (jnp.sqrt(jnp.clip(d2, 1e-12, None))
          + jnp.where(masked, 1.0e8, 0.0))
    d_i = jnp.sum(d5, axis=1, keepdims=True) * (1.0 / k)

    q = gtc_ref[...]
    br = q.shape[0]
    rowid = i * br + jax.lax.broadcasted_iota(jnp.int32, (br, 1), 0)
    rmask = rowid < n_valid
    s_center, s_ckl, s_skl = _loss_tail(
        q, mu_ref[...], bls_ref[...], pos_ref[...], stride_ref[...],
        d_i, sig_ref[0, 0], rmask)
    _accum_out(i, out_ref, s_center, s_ckl, s_skl)


def _pick_br(n):
    for br in (400, 256, 128, 64, 32, 16, 8):
        if n % br == 0:
            return br
    return n


def _pad_rows(a, n_to):
    n = a.shape[0]
    return jnp.pad(a, ((0, n_to - n),) + ((0, 0),) * (a.ndim - 1))


@jax.jit
def kernel(bbox_mu, bbox_log_sigma, pos_points, pos_strides, gt_centers,
           gt_centers_list, cur_iter):
    n = bbox_mu.shape[0]
    keys = gt_centers_list.reshape(-1, 2)
    m = keys.shape[0]
    k = min(_KNN_K, m - 1)

    ratio = jnp.clip((cur_iter - _WARMUP_ITERS) / max(_ANNEAL_ITERS, 1), 0.0, 1.0)
    eff_lkl = _LAMBDA_KL_WARMUP + ratio * (_LAMBDA_KL - _LAMBDA_KL_WARMUP)
    sigma_s = _SIGMA_S_INIT - ratio * (_SIGMA_S_INIT - _SIGMA_S_FINAL)
    sig_eff = jnp.maximum(sigma_s, 1.0).astype(jnp.float32).reshape(1, 1)

    stride_all = pos_strides.astype(jnp.float32).reshape(n, 1)

    _BR = 400
    nb_all = (n + _BR - 1) // _BR
    nb_tc = max(1, min(nb_all, int(round(n * 0.60 / _BR))))
    n_tc = min(nb_tc * _BR, n)
    n_sc = n - n_tc
    nb_b = (n_sc + _BR - 1) // _BR
    n_row_pad = nb_all * _BR

    def padr(a, cval=0.0):
        if n_row_pad == n:
            return a
        return jnp.pad(a, ((0, n_row_pad - n),) + ((0, 0),) * (a.ndim - 1),
                       constant_values=cval)

    mu_p = padr(bbox_mu)
    bls_p = padr(bbox_log_sigma)
    pos_p = padr(pos_points)
    str_p = padr(stride_all, 1.0)
    gtc_p = padr(gt_centers)

    keysT = keys.T

    if n_sc > 0:
        n_sc_pad = ((n_sc + _CHUNK - 1) // _CHUNK) * _CHUNK
        qx = jnp.pad(gt_centers[n_tc:, 0], (0, n_sc_pad - n_sc),
                     constant_values=1.0e9)
        qy = jnp.pad(gt_centers[n_tc:, 1], (0, n_sc_pad - n_sc),
                     constant_values=1.0e9)
        m_pad = ((m + _L - 1) // _L) * _L
        kx = jnp.pad(keys[:, 0], (0, m_pad - m), constant_values=1.0e8)
        ky = jnp.pad(keys[:, 1], (0, m_pad - m), constant_values=1.0e8)
        knn_s = _knn_sc(qx, qy, kx, ky, k)
        knn_rows = knn_s.reshape(k, n_sc_pad).T
        if nb_b * _BR > n_sc_pad:
            knn_rows = _pad_rows(knn_rows, nb_b * _BR)

    row_spec = lambda c: pl.BlockSpec((_BR, c), lambda i: (i, 0))
    full_spec = lambda r, c: pl.BlockSpec((r, c), lambda i: (0, 0))

    out_a = pl.pallas_call(
        functools.partial(_tc_body, m=m, k=k, n_valid=n_tc),
        grid=(nb_tc,),
        in_specs=[
            row_spec(4),
            row_spec(4),
            row_spec(2),
            row_spec(1),
            row_spec(2),
            full_spec(2, m),
            full_spec(1, 1),
        ],
        out_specs=pl.BlockSpec((1, 128), lambda i: (0, 0)),
        out_shape=jax.ShapeDtypeStruct((1, 128), jnp.float32),
    )(mu_p, bls_p, pos_p, str_p, gtc_p, keysT, sig_eff)

    s_center = out_a[0, 0]
    s_ckl = out_a[0, 1]
    s_skl = out_a[0, 2]

    if n_sc > 0:
        tail_spec = lambda c: pl.BlockSpec((_BR, c), lambda i: (i + nb_tc, 0))
        out_b = pl.pallas_call(
            functools.partial(_sc_loss_body, k=k, n_valid=n_sc),
            grid=(nb_b,),
            in_specs=[
                tail_spec(4),
                tail_spec(4),
                tail_spec(2),
                tail_spec(1),
                tail_spec(2),
                pl.BlockSpec((_BR, k), lambda i: (i, 0)),
                pl.BlockSpec((1, 1), lambda i: (0, 0)),
            ],
            out_specs=pl.BlockSpec((1, 128), lambda i: (0, 0)),
            out_shape=jax.ShapeDtypeStruct((1, 128), jnp.float32),
        )(mu_p, bls_p, pos_p, str_p, gtc_p, knn_rows, sig_eff)
        s_center = s_center + out_b[0, 0]
        s_ckl = s_ckl + out_b[0, 1]
        s_skl = s_skl + out_b[0, 2]

    l_center = s_center / n
    center_kl = s_ckl / n
    scale_kl = s_skl / n
    l_kl = center_kl + ratio * scale_kl
    weighted_center = (_LAMBDA_CENTER * l_center).astype(jnp.float32)
    weighted_kl = (eff_lkl * l_kl).astype(jnp.float32)
    return (weighted_center, weighted_kl)


# speedup vs baseline: 1.0926x; 1.0357x over previous
import functools

import jax
import jax.numpy as jnp
from jax import lax
from jax.experimental import pallas as pl
from jax.experimental.pallas import tpu as pltpu
from jax.experimental.pallas import tpu_sc as plsc

_LAMBDA_CENTER = 1.0
_LAMBDA_KL = 0.05
_LAMBDA_KL_WARMUP = 0.005
_KNN_K = 5
_SIGMA_S_INIT = 2.0
_SIGMA_S_FINAL = 0.8
_WARMUP_ITERS = 1000
_ANNEAL_ITERS = 3000
_PRIOR_DELTA_MIN = 0.5
_PRIOR_DELTA_MAX = 20.0
_LOG_SIGMA_MIN = -6.0
_LOG_SIGMA_MAX = 4.0
_BIG = 3.0e38

_NC, _NS, _L = 2, 16, 16
_NW = _NC * _NS
_CHUNK = _NW * _L
_MASK_PEN = 1.0e10


def _knn_sc(qx, qy, kx, ky, k):
    n_pad = qx.shape[0]
    m = kx.shape[0]
    rpt = n_pad // _NW
    nv = rpt // _L

    mesh = plsc.VectorSubcoreMesh(core_axis_name="c", subcore_axis_name="s",
                                  num_cores=_NC, num_subcores=_NS)

    @functools.partial(
        pl.kernel,
        out_type=jax.ShapeDtypeStruct((k * n_pad,), jnp.float32),
        mesh=mesh,
        scratch_types=[
            pltpu.VMEM((rpt,), jnp.float32),
            pltpu.VMEM((rpt,), jnp.float32),
            pltpu.VMEM((m,), jnp.float32),
            pltpu.VMEM((m,), jnp.float32),
            pltpu.VMEM((k * rpt,), jnp.float32),
        ],
    )
    def knn(qx_h, qy_h, kx_h, ky_h, out_h, qx_v, qy_v, kx_v, ky_v, out_v):
        wid = lax.axis_index("s") * _NC + lax.axis_index("c")
        base = wid * rpt
        pltpu.sync_copy(qx_h.at[pl.ds(base, rpt)], qx_v)
        pltpu.sync_copy(qy_h.at[pl.ds(base, rpt)], qy_v)
        pltpu.sync_copy(kx_h, kx_v)
        pltpu.sync_copy(ky_h, ky_v)
        for r in range(nv):
            qxv = qx_v[pl.ds(r * _L, _L)]
            qyv = qy_v[pl.ds(r * _L, _L)]
            init = tuple(jnp.full((_L,), _BIG, jnp.float32) for _ in range(k))

            def body(c, ms, qxv=qxv, qyv=qyv):
                kxv16 = kx_v[pl.ds(c * _L, _L)]
                kyv16 = ky_v[pl.ds(c * _L, _L)]
                for l in range(_L):
                    dx = qxv - kxv16[l]
                    dy = qyv - kyv16[l]
                    d2 = dx * dx + dy * dy
                    t = jnp.where(d2 < 1e-4, d2 + _MASK_PEN, d2)
                    out = []
                    for mm in ms:
                        out.append(jnp.minimum(mm, t))
                        t = jnp.maximum(mm, t)
                    ms = tuple(out)
                return ms

            ms = lax.fori_loop(0, m // _L, body, init)
            for jj in range(k):
                out_v[pl.ds(jj * rpt + r * _L, _L)] = ms[jj]
        for jj in range(k):
            pltpu.sync_copy(out_v.at[pl.ds(jj * rpt, rpt)],
                            out_h.at[pl.ds(jj * n_pad + base, rpt)])

    return knn(qx, qy, kx, ky)


def _loss_tail(q, mu, bls, pos, stride, d_i, sig_s, rmask):
    gt_delta = (q - pos) / stride
    diff = mu[:, 0:2] - gt_delta
    ad = jnp.abs(diff)
    sl1 = jnp.where(ad < 1.0, 0.5 * diff * diff, ad - 0.5)

    d_norm = jnp.clip(d_i / stride, _PRIOR_DELTA_MIN, _PRIOR_DELTA_MAX)
    sigma_c = jnp.maximum(d_norm, 1.0)
    mu_s = jnp.log(d_norm)

    log_sq = jnp.clip(bls, _LOG_SIGMA_MIN, _LOG_SIGMA_MAX)
    sigma_q = jnp.exp(log_sq)

    prior_mu = jnp.concatenate(
        [jnp.zeros_like(d_norm), jnp.zeros_like(d_norm), mu_s, mu_s], axis=1)
    prior_sigma = jnp.concatenate(
        [sigma_c, sigma_c,
         jnp.full_like(d_norm, 1.0) * sig_s,
         jnp.full_like(d_norm, 1.0) * sig_s], axis=1)
    sigma_p = jnp.clip(prior_sigma, 0.0001, None)

    dm = mu - prior_mu
    kl = (jnp.log(sigma_p / sigma_q)
          + (sigma_q * sigma_q + dm * dm) / (2.0 * sigma_p * sigma_p) - 0.5)

    s_center = jnp.sum(jnp.where(rmask, sl1[:, 0:1] + sl1[:, 1:2], 0.0))
    s_ckl = jnp.sum(jnp.where(rmask, kl[:, 0:1] + kl[:, 1:2], 0.0))
    s_skl = jnp.sum(jnp.where(rmask, kl[:, 2:3] + kl[:, 3:4], 0.0))
    return s_center, s_ckl, s_skl


def _accum_out(i, out_ref, s_center, s_ckl, s_skl):
    lane = jax.lax.broadcasted_iota(jnp.int32, (1, 128), 1)
    vec = (jnp.where(lane == 0, s_center, 0.0)
           + jnp.where(lane == 1, s_ckl, 0.0)
           + jnp.where(lane == 2, s_skl, 0.0))

    @pl.when(i == 0)
    def _():
        out_ref[...] = jnp.zeros_like(out_ref)

    out_ref[...] += vec


def _tc_body(slab_ref, keys_ref, sig_ref, out_ref, *, m, k, n_valid):
    i = pl.program_id(0)

    slab = slab_ref[...]
    q = slab[:, 10:12]
    qx = q[:, 0:1]
    qy = q[:, 1:2]
    kx = keys_ref[0:1, :]
    ky = keys_ref[1:2, :]

    qn = qx * qx + qy * qy
    kn = kx * kx + ky * ky
    cross = qx * kx + qy * ky
    d2 = qn + kn - 2.0 * cross
    t = jnp.where(d2 < 1e-4, d2 + _MASK_PEN, d2)

    total = jnp.zeros((t.shape[0], 1), jnp.float32)
    remaining = jnp.full((t.shape[0], 1), float(k), jnp.float32)
    for r in range(k):
        mn = jnp.min(t, axis=1, keepdims=True)
        msk = t == mn
        c = jnp.sum(msk, axis=1, keepdims=True).astype(jnp.float32)
        take = jnp.minimum(c, remaining)
        pen = mn >= 1.0e9
        mn2 = mn - jnp.where(pen, _MASK_PEN, 0.0)
        dval = (jnp.sqrt(jnp.clip(mn2, 1e-12, None))
                + jnp.where(pen, 1.0e8, 0.0))
        total = total + dval * take
        remaining = remaining - take
        if r < k - 1:
            t = jnp.where(msk, _BIG, t)
    d_i = total * (1.0 / k)

    br = q.shape[0]
    rowid = i * br + jax.lax.broadcasted_iota(jnp.int32, (br, 1), 0)
    rmask = rowid < n_valid
    s_center, s_ckl, s_skl = _loss_tail(
        q, slab[:, 0:4], slab[:, 4:8], slab[:, 8:10], slab[:, 12:13],
        d_i, sig_ref[0, 0], rmask)
    _accum_out(i, out_ref, s_center, s_ckl, s_skl)


def _sc_loss_body(slab_ref, knn_ref, sig_ref, out_ref, *, k, n_valid):
    i = pl.program_id(0)

    s5 = knn_ref[...]
    masked = s5 >= 1.0e9
    d2 = s5 - jnp.where(masked, _MASK_PEN, 0.0)
    d5 = (jnp.sqrt(jnp.clip(d2, 1e-12, None))
          + jnp.where(masked, 1.0e8, 0.0))
    d_i = jnp.sum(d5, axis=1, keepdims=True) * (1.0 / k)

    slab = slab_ref[...]
    q = slab[:, 10:12]
    br = q.shape[0]
    rowid = i * br + jax.lax.broadcasted_iota(jnp.int32, (br, 1), 0)
    rmask = rowid < n_valid
    s_center, s_ckl, s_skl = _loss_tail(
        q, slab[:, 0:4], slab[:, 4:8], slab[:, 8:10], slab[:, 12:13],
        d_i, sig_ref[0, 0], rmask)
    _accum_out(i, out_ref, s_center, s_ckl, s_skl)


def _pick_br(n):
    for br in (400, 256, 128, 64, 32, 16, 8):
        if n % br == 0:
            return br
    return n


def _pad_rows(a, n_to):
    n = a.shape[0]
    return jnp.pad(a, ((0, n_to - n),) + ((0, 0),) * (a.ndim - 1))


@jax.jit
def kernel(bbox_mu, bbox_log_sigma, pos_points, pos_strides, gt_centers,
           gt_centers_list, cur_iter):
    n = bbox_mu.shape[0]
    keys = gt_centers_list.reshape(-1, 2)
    m = keys.shape[0]
    k = min(_KNN_K, m - 1)

    ratio = jnp.clip((cur_iter - _WARMUP_ITERS) / max(_ANNEAL_ITERS, 1), 0.0, 1.0)
    eff_lkl = _LAMBDA_KL_WARMUP + ratio * (_LAMBDA_KL - _LAMBDA_KL_WARMUP)
    sigma_s = _SIGMA_S_INIT - ratio * (_SIGMA_S_INIT - _SIGMA_S_FINAL)
    sig_eff = jnp.maximum(sigma_s, 1.0).astype(jnp.float32).reshape(1, 1)

    stride_all = pos_strides.astype(jnp.float32).reshape(n, 1)

    _BR = 400
    nb_all = (n + _BR - 1) // _BR
    nb_tc = max(1, min(nb_all, int(round(n * 0.56 / _BR))))
    n_tc = min(nb_tc * _BR, n)
    n_sc = n - n_tc
    nb_b = (n_sc + _BR - 1) // _BR
    n_row_pad = nb_all * _BR

    slab = jnp.concatenate(
        [bbox_mu, bbox_log_sigma, pos_points, gt_centers, stride_all], axis=1)
    if n_row_pad != n:
        slab = jnp.pad(slab, ((0, n_row_pad - n), (0, 0)))

    keysT = keys.T

    if n_sc > 0:
        n_sc_pad = ((n_sc + _CHUNK - 1) // _CHUNK) * _CHUNK
        qx = jnp.pad(gt_centers[n_tc:, 0], (0, n_sc_pad - n_sc),
                     constant_values=1.0e9)
        qy = jnp.pad(gt_centers[n_tc:, 1], (0, n_sc_pad - n_sc),
                     constant_values=1.0e9)
        m_pad = ((m + _L - 1) // _L) * _L
        kx = jnp.pad(keys[:, 0], (0, m_pad - m), constant_values=1.0e8)
        ky = jnp.pad(keys[:, 1], (0, m_pad - m), constant_values=1.0e8)
        knn_s = _knn_sc(qx, qy, kx, ky, k)
        knn_rows = knn_s.reshape(k, n_sc_pad).T
        if nb_b * _BR > n_sc_pad:
            knn_rows = _pad_rows(knn_rows, nb_b * _BR)

    row_spec = lambda c: pl.BlockSpec((_BR, c), lambda i: (i, 0))
    full_spec = lambda r, c: pl.BlockSpec((r, c), lambda i: (0, 0))

    out_a = pl.pallas_call(
        functools.partial(_tc_body, m=m, k=k, n_valid=n_tc),
        grid=(nb_tc,),
        in_specs=[
            row_spec(13),
            full_spec(2, m),
            full_spec(1, 1),
        ],
        out_specs=pl.BlockSpec((1, 128), lambda i: (0, 0)),
        out_shape=jax.ShapeDtypeStruct((1, 128), jnp.float32),
    )(slab, keysT, sig_eff)

    s_center = out_a[0, 0]
    s_ckl = out_a[0, 1]
    s_skl = out_a[0, 2]

    if n_sc > 0:
        out_b = pl.pallas_call(
            functools.partial(_sc_loss_body, k=k, n_valid=n_sc),
            grid=(nb_b,),
            in_specs=[
                pl.BlockSpec((_BR, 13), lambda i: (i + nb_tc, 0)),
                pl.BlockSpec((_BR, k), lambda i: (i, 0)),
                pl.BlockSpec((1, 1), lambda i: (0, 0)),
            ],
            out_specs=pl.BlockSpec((1, 128), lambda i: (0, 0)),
            out_shape=jax.ShapeDtypeStruct((1, 128), jnp.float32),
        )(slab, knn_rows, sig_eff)
        s_center = s_center + out_b[0, 0]
        s_ckl = s_ckl + out_b[0, 1]
        s_skl = s_skl + out_b[0, 2]

    l_center = s_center / n
    center_kl = s_ckl / n
    scale_kl = s_skl / n
    l_kl = center_kl + ratio * scale_kl
    weighted_center = (_LAMBDA_CENTER * l_center).astype(jnp.float32)
    weighted_kl = (eff_lkl * l_kl).astype(jnp.float32)
    return (weighted_center, weighted_kl)


# speedup vs baseline: 1.1055x; 1.0117x over previous
import functools

import jax
import jax.numpy as jnp
from jax import lax
from jax.experimental import pallas as pl
from jax.experimental.pallas import tpu as pltpu
from jax.experimental.pallas import tpu_sc as plsc

_LAMBDA_CENTER = 1.0
_LAMBDA_KL = 0.05
_LAMBDA_KL_WARMUP = 0.005
_KNN_K = 5
_SIGMA_S_INIT = 2.0
_SIGMA_S_FINAL = 0.8
_WARMUP_ITERS = 1000
_ANNEAL_ITERS = 3000
_PRIOR_DELTA_MIN = 0.5
_PRIOR_DELTA_MAX = 20.0
_LOG_SIGMA_MIN = -6.0
_LOG_SIGMA_MAX = 4.0
_BIG = 3.0e38

_NC, _NS, _L = 2, 16, 16
_NW = _NC * _NS
_CHUNK = _NW * _L
_MASK_PEN = 1.0e10


def _knn_sc(qx, qy, kx, ky, k):
    n_pad = qx.shape[0]
    m = kx.shape[0]
    rpt = n_pad // _NW
    nv = rpt // _L

    mesh = plsc.VectorSubcoreMesh(core_axis_name="c", subcore_axis_name="s",
                                  num_cores=_NC, num_subcores=_NS)

    @functools.partial(
        pl.kernel,
        out_type=jax.ShapeDtypeStruct((k * n_pad,), jnp.float32),
        mesh=mesh,
        scratch_types=[
            pltpu.VMEM((rpt,), jnp.float32),
            pltpu.VMEM((rpt,), jnp.float32),
            pltpu.VMEM((m,), jnp.float32),
            pltpu.VMEM((m,), jnp.float32),
            pltpu.VMEM((k * rpt,), jnp.float32),
        ],
    )
    def knn(qx_h, qy_h, kx_h, ky_h, out_h, qx_v, qy_v, kx_v, ky_v, out_v):
        wid = lax.axis_index("s") * _NC + lax.axis_index("c")
        base = wid * rpt
        pltpu.sync_copy(qx_h.at[pl.ds(base, rpt)], qx_v)
        pltpu.sync_copy(qy_h.at[pl.ds(base, rpt)], qy_v)
        pltpu.sync_copy(kx_h, kx_v)
        pltpu.sync_copy(ky_h, ky_v)
        for r in range(nv):
            qxv = qx_v[pl.ds(r * _L, _L)]
            qyv = qy_v[pl.ds(r * _L, _L)]
            init = tuple(jnp.full((_L,), _BIG, jnp.float32) for _ in range(k))

            def body(c, ms, qxv=qxv, qyv=qyv):
                kxv16 = kx_v[pl.ds(c * _L, _L)]
                kyv16 = ky_v[pl.ds(c * _L, _L)]
                for l in range(_L):
                    dx = qxv - kxv16[l]
                    dy = qyv - kyv16[l]
                    d2 = dx * dx + dy * dy
                    t = jnp.where(d2 < 1e-4, d2 + _MASK_PEN, d2)
                    out = []
                    for jj, mm in enumerate(ms):
                        out.append(jnp.minimum(mm, t))
                        if jj < len(ms) - 1:
                            t = jnp.maximum(mm, t)
                    ms = tuple(out)
                return ms

            ms = lax.fori_loop(0, m // _L, body, init)
            for jj in range(k):
                out_v[pl.ds(jj * rpt + r * _L, _L)] = ms[jj]
        for jj in range(k):
            pltpu.sync_copy(out_v.at[pl.ds(jj * rpt, rpt)],
                            out_h.at[pl.ds(jj * n_pad + base, rpt)])

    return knn(qx, qy, kx, ky)


def _loss_tail(q, mu, bls, pos, stride, d_i, sig_s, rmask):
    gt_delta = (q - pos) / stride
    diff = mu[:, 0:2] - gt_delta
    ad = jnp.abs(diff)
    sl1 = jnp.where(ad < 1.0, 0.5 * diff * diff, ad - 0.5)

    d_norm = jnp.clip(d_i / stride, _PRIOR_DELTA_MIN, _PRIOR_DELTA_MAX)
    sigma_c = jnp.maximum(d_norm, 1.0)
    mu_s = jnp.log(d_norm)

    log_sq = jnp.clip(bls, _LOG_SIGMA_MIN, _LOG_SIGMA_MAX)
    sigma_q = jnp.exp(log_sq)

    prior_mu = jnp.concatenate(
        [jnp.zeros_like(d_norm), jnp.zeros_like(d_norm), mu_s, mu_s], axis=1)
    prior_sigma = jnp.concatenate(
        [sigma_c, sigma_c,
         jnp.full_like(d_norm, 1.0) * sig_s,
         jnp.full_like(d_norm, 1.0) * sig_s], axis=1)
    sigma_p = jnp.clip(prior_sigma, 0.0001, None)

    dm = mu - prior_mu
    kl = (jnp.log(sigma_p / sigma_q)
          + (sigma_q * sigma_q + dm * dm) / (2.0 * sigma_p * sigma_p) - 0.5)

    s_center = jnp.sum(jnp.where(rmask, sl1[:, 0:1] + sl1[:, 1:2], 0.0))
    s_ckl = jnp.sum(jnp.where(rmask, kl[:, 0:1] + kl[:, 1:2], 0.0))
    s_skl = jnp.sum(jnp.where(rmask, kl[:, 2:3] + kl[:, 3:4], 0.0))
    return s_center, s_ckl, s_skl


def _accum_out(i, out_ref, s_center, s_ckl, s_skl):
    lane = jax.lax.broadcasted_iota(jnp.int32, (1, 128), 1)
    vec = (jnp.where(lane == 0, s_center, 0.0)
           + jnp.where(lane == 1, s_ckl, 0.0)
           + jnp.where(lane == 2, s_skl, 0.0))

    @pl.when(i == 0)
    def _():
        out_ref[...] = jnp.zeros_like(out_ref)

    out_ref[...] += vec


def _tc_body(slab_ref, keys_ref, sig_ref, out_ref, *, m, k, n_valid):
    i = pl.program_id(0)

    slab = slab_ref[...]
    q = slab[:, 10:12]
    qx = q[:, 0:1]
    qy = q[:, 1:2]
    kx = keys_ref[0:1, :]
    ky = keys_ref[1:2, :]

    qn = qx * qx + qy * qy
    kn = kx * kx + ky * ky
    cross = qx * kx + qy * ky
    d2 = qn + kn - 2.0 * cross
    t = jnp.where(d2 < 1e-4, d2 + _MASK_PEN, d2)

    total = jnp.zeros((t.shape[0], 1), jnp.float32)
    remaining = jnp.full((t.shape[0], 1), float(k), jnp.float32)
    for r in range(k):
        mn = jnp.min(t, axis=1, keepdims=True)
        msk = t == mn
        c = jnp.sum(msk, axis=1, keepdims=True).astype(jnp.float32)
        take = jnp.minimum(c, remaining)
        pen = mn >= 1.0e9
        mn2 = mn - jnp.where(pen, _MASK_PEN, 0.0)
        dval = (jnp.sqrt(jnp.clip(mn2, 1e-12, None))
                + jnp.where(pen, 1.0e8, 0.0))
        total = total + dval * take
        remaining = remaining - take
        if r < k - 1:
            t = jnp.where(msk, _BIG, t)
    d_i = total * (1.0 / k)

    br = q.shape[0]
    rowid = i * br + jax.lax.broadcasted_iota(jnp.int32, (br, 1), 0)
    rmask = rowid < n_valid
    s_center, s_ckl, s_skl = _loss_tail(
        q, slab[:, 0:4], slab[:, 4:8], slab[:, 8:10], slab[:, 12:13],
        d_i, sig_ref[0, 0], rmask)
    _accum_out(i, out_ref, s_center, s_ckl, s_skl)


def _sc_loss_body(slab_ref, knn_ref, sig_ref, out_ref, *, k, n_valid):
    i = pl.program_id(0)

    s5 = knn_ref[...]
    masked = s5 >= 1.0e9
    d2 = s5 - jnp.where(masked, _MASK_PEN, 0.0)
    d5 = (jnp.sqrt(jnp.clip(d2, 1e-12, None))
          + jnp.where(masked, 1.0e8, 0.0))
    d_i = jnp.sum(d5, axis=1, keepdims=True) * (1.0 / k)

    slab = slab_ref[...]
    q = slab[:, 10:12]
    br = q.shape[0]
    rowid = i * br + jax.lax.broadcasted_iota(jnp.int32, (br, 1), 0)
    rmask = rowid < n_valid
    s_center, s_ckl, s_skl = _loss_tail(
        q, slab[:, 0:4], slab[:, 4:8], slab[:, 8:10], slab[:, 12:13],
        d_i, sig_ref[0, 0], rmask)
    _accum_out(i, out_ref, s_center, s_ckl, s_skl)


def _pick_br(n):
    for br in (400, 256, 128, 64, 32, 16, 8):
        if n % br == 0:
            return br
    return n


def _pad_rows(a, n_to):
    n = a.shape[0]
    return jnp.pad(a, ((0, n_to - n),) + ((0, 0),) * (a.ndim - 1))


@jax.jit
def kernel(bbox_mu, bbox_log_sigma, pos_points, pos_strides, gt_centers,
           gt_centers_list, cur_iter):
    n = bbox_mu.shape[0]
    keys = gt_centers_list.reshape(-1, 2)
    m = keys.shape[0]
    k = min(_KNN_K, m - 1)

    ratio = jnp.clip((cur_iter - _WARMUP_ITERS) / max(_ANNEAL_ITERS, 1), 0.0, 1.0)
    eff_lkl = _LAMBDA_KL_WARMUP + ratio * (_LAMBDA_KL - _LAMBDA_KL_WARMUP)
    sigma_s = _SIGMA_S_INIT - ratio * (_SIGMA_S_INIT - _SIGMA_S_FINAL)
    sig_eff = jnp.maximum(sigma_s, 1.0).astype(jnp.float32).reshape(1, 1)

    stride_all = pos_strides.astype(jnp.float32).reshape(n, 1)

    _BR = 400
    nb_all = (n + _BR - 1) // _BR
    nb_tc = max(1, min(nb_all, int(round(n * 0.56 / _BR))))
    n_tc = min(nb_tc * _BR, n)
    n_sc = n - n_tc
    nb_b = (n_sc + _BR - 1) // _BR
    n_row_pad = nb_all * _BR

    slab = jnp.concatenate(
        [bbox_mu, bbox_log_sigma, pos_points, gt_centers, stride_all], axis=1)
    if n_row_pad != n:
        slab = jnp.pad(slab, ((0, n_row_pad - n), (0, 0)))

    keysT = keys.T

    if n_sc > 0:
        n_sc_pad = ((n_sc + _CHUNK - 1) // _CHUNK) * _CHUNK
        qx = jnp.pad(gt_centers[n_tc:, 0], (0, n_sc_pad - n_sc),
                     constant_values=1.0e9)
        qy = jnp.pad(gt_centers[n_tc:, 1], (0, n_sc_pad - n_sc),
                     constant_values=1.0e9)
        m_pad = ((m + _L - 1) // _L) * _L
        kx = jnp.pad(keys[:, 0], (0, m_pad - m), constant_values=1.0e8)
        ky = jnp.pad(keys[:, 1], (0, m_pad - m), constant_values=1.0e8)
        knn_s = _knn_sc(qx, qy, kx, ky, k)
        knn_rows = knn_s.reshape(k, n_sc_pad).T

    row_spec = lambda c: pl.BlockSpec((_BR, c), lambda i: (i, 0))
    full_spec = lambda r, c: pl.BlockSpec((r, c), lambda i: (0, 0))

    out_a = pl.pallas_call(
        functools.partial(_tc_body, m=m, k=k, n_valid=n_tc),
        grid=(nb_tc,),
        in_specs=[
            row_spec(13),
            full_spec(2, m),
            full_spec(1, 1),
        ],
        out_specs=pl.BlockSpec((1, 128), lambda i: (0, 0)),
        out_shape=jax.ShapeDtypeStruct((1, 128), jnp.float32),
    )(slab, keysT, sig_eff)

    s_center = out_a[0, 0]
    s_ckl = out_a[0, 1]
    s_skl = out_a[0, 2]

    if n_sc > 0:
        slab_b = jnp.pad(slab[n_tc:n_row_pad],
                         ((0, n_sc_pad - (n_row_pad - n_tc)), (0, 0)))
        out_b = pl.pallas_call(
            functools.partial(_sc_loss_body, k=k, n_valid=n_sc),
            grid=(1,),
            in_specs=[
                pl.BlockSpec((n_sc_pad, 13), lambda i: (0, 0)),
                pl.BlockSpec((n_sc_pad, k), lambda i: (0, 0)),
                pl.BlockSpec((1, 1), lambda i: (0, 0)),
            ],
            out_specs=pl.BlockSpec((1, 128), lambda i: (0, 0)),
            out_shape=jax.ShapeDtypeStruct((1, 128), jnp.float32),
        )(slab_b, knn_rows, sig_eff)
        s_center = s_center + out_b[0, 0]
        s_ckl = s_ckl + out_b[0, 1]
        s_skl = s_skl + out_b[0, 2]

    l_center = s_center / n
    center_kl = s_ckl / n
    scale_kl = s_skl / n
    l_kl = center_kl + ratio * scale_kl
    weighted_center = (_LAMBDA_CENTER * l_center).astype(jnp.float32)
    weighted_kl = (eff_lkl * l_kl).astype(jnp.float32)
    return (weighted_center, weighted_kl)
